# baseline (speedup 1.0000x reference)
import jax
import jax.numpy as jnp
from jax import lax
from jax.experimental import pallas as pl
from jax.experimental.pallas import tpu as pltpu

ABLATION = 10000000.0

B = 64
C = 512
HW = 784
G = 56


def _min_body(x_ref, vals_ref, macc):
    i = pl.program_id(0)
    bmin = jnp.min(x_ref[...])

    @pl.when(i == 0)
    def _():
        macc[0] = bmin

    @pl.when(i > 0)
    def _():
        macc[0] = jnp.minimum(macc[0], bmin)

    @pl.when(i == pl.num_programs(0) - 1)
    def _():
        m0 = macc[0]
        it = lax.broadcasted_iota(jnp.int32, (B, 1), 0)

        def step(t, s):
            fs = jnp.where(s == 0.0, 0.0, s - ABLATION)
            return jnp.where(it >= t, fs, s)

        vals_ref[...] = lax.fori_loop(0, B, step, jnp.full((B, 1), m0, jnp.float32))


def _min_pass(xt):
    return pl.pallas_call(
        _min_body,
        grid=(HW // G,),
        in_specs=[pl.BlockSpec((G, B, C), lambda i: (i, 0, 0))],
        out_specs=pl.BlockSpec((B, 1), lambda i: (0, 0)),
        out_shape=jax.ShapeDtypeStruct((B, 1), jnp.float32),
        scratch_shapes=[pltpu.SMEM((1,), jnp.float32)],
    )(xt)


def _apply_body(x_ref, vals_ref, idx_ref, y_ref):
    lane = lax.broadcasted_iota(jnp.int32, (1, B, C), 2)
    eq = lane == idx_ref[...].reshape(1, B, 1)
    vb = jnp.broadcast_to(vals_ref[...].reshape(1, B, 1), (1, B, C))
    y_ref[...] = jnp.where(eq, vb, x_ref[...])


def _apply_pass(xt, vals, idx):
    return pl.pallas_call(
        _apply_body,
        grid=(HW // G,),
        in_specs=[
            pl.BlockSpec((G, B, C), lambda i: (i, 0, 0)),
            pl.BlockSpec((B, 1), lambda i: (0, 0)),
            pl.BlockSpec((B, 1), lambda i: (0, 0)),
        ],
        out_specs=pl.BlockSpec((G, B, C), lambda i: (i, 0, 0)),
        out_shape=jax.ShapeDtypeStruct((HW, B, C), jnp.float32),
    )(xt, vals, idx)


@jax.jit
def kernel(x, indices):
    xt = x.transpose(2, 3, 0, 1).reshape(HW, B, C)
    vals = _min_pass(xt)
    yt = _apply_pass(xt, vals, indices.reshape(B, 1))
    return yt.reshape(28, 28, B, C).transpose(2, 3, 0, 1)
